# baseline (speedup 1.0000x reference)
import functools
import math

import jax
import jax.numpy as jnp
from jax import lax
from jax.experimental import pallas as pl
from jax.experimental.pallas import tpu as pltpu
from jax.experimental.pallas import tpu_sc as plsc

SEQ = 200
DIM = 32
VOCAB = 1000000
LANES = 16
NC, NS = 2, 16
NW = NC * NS
SCALE = float(math.sqrt(float(DIM)))

QUART = 256000
R0 = 2048
NQB = QUART // R0
VPAD = 4 * QUART
BBLK = 128
GRP = 2
UNIT = GRP * BBLK
TPAD = 133
NBUF = 5


def _tc_transpose(t0, t1, t2, t3, out_ref):
    out_ref[...] = jnp.concatenate(
        [t0[...].T, t1[...].T, t2[...].T, t3[...].T], axis=1)


def _convert_table(token_table):
    tabt = token_table.T
    conv = pl.pallas_call(
        _tc_transpose,
        grid=(NQB,),
        in_specs=[
            pl.BlockSpec((DIM, R0), functools.partial(
                lambda j, i: (0, jnp.minimum(j * NQB + i, VOCAB // R0)), j))
            for j in range(4)
        ],
        out_specs=pl.BlockSpec((R0, 128), lambda i: (i, 0)),
        out_shape=jax.ShapeDtypeStruct((QUART, 128), jnp.float32),
    )(tabt, tabt, tabt, tabt)
    return conv.reshape(VPAD, DIM)


def _sc_body(idx_hbm, tok_hbm, pos_hbm, out_hbm,
             pos_v, idxbufs, rowbufs, tilebufs, isems, gsems, osems,
             *, units_per_w):
    wid = lax.axis_index("s") * NC + lax.axis_index("c")
    ubase = wid * units_per_w
    pltpu.sync_copy(pos_hbm, pos_v)
    iota16 = lax.iota(jnp.int32, 16)

    nq = 4096 // UNIT

    def unit_sb(u):
        g = ubase + u
        return g // nq, g % nq

    def start_idx(u, b):
        s, q = unit_sb(u)
        pltpu.async_copy(idx_hbm.at[s, pl.ds(q * UNIT, UNIT)], idxbufs[b],
                         isems[b])

    def wait_idx(u, b):
        s, q = unit_sb(u)
        pltpu.make_async_copy(idx_hbm.at[s, pl.ds(q * UNIT, UNIT)],
                              idxbufs[b], isems[b]).wait()

    def remap_idx(b):
        buf = idxbufs[b]
        for k in range(UNIT // LANES):
            v = buf[pl.ds(k * LANES, LANES)]
            vsub = ((v >= QUART).astype(jnp.int32)
                    + (v >= 2 * QUART).astype(jnp.int32)
                    + (v >= 3 * QUART).astype(jnp.int32))
            buf[pl.ds(k * LANES, LANES)] = (v - vsub * QUART) * 4 + vsub

    def start_gather(b):
        pltpu.async_copy(tok_hbm.at[idxbufs[b]], rowbufs[b], gsems[b])

    def wait_gather(b):
        pltpu.make_async_copy(tok_hbm.at[idxbufs[b]], rowbufs[b],
                              gsems[b]).wait()

    def start_out(u, b):
        s, q = unit_sb(u)
        pltpu.async_copy(tilebufs[b].at[:, :, :, pl.ds(0, BBLK)],
                         out_hbm.at[s, :, pl.ds(q * GRP, GRP)], osems[b])

    def wait_out(u, b):
        s, q = unit_sb(u)
        pltpu.make_async_copy(tilebufs[b].at[:, :, :, pl.ds(0, BBLK)],
                              out_hbm.at[s, :, pl.ds(q * GRP, GRP)],
                              osems[b]).wait()

    dvec = [iota16 + (h * LANES) for h in range(2)]
    dgvec = [d // 8 for d in dvec]
    drvec = [d % 8 for d in dvec]
    gconst = [jnp.broadcast_to(jnp.int32(g), (16,)) for g in range(GRP)]

    def compute(u, b):
        s, _ = unit_sb(u)
        rows = rowbufs[b]
        tile = tilebufs[b]
        poss = [pos_v[s, pl.ds(h * LANES, LANES)] for h in range(2)]

        for g in range(GRP):
            def t_body(i, _, g=g):
                for j in range(4):
                    t2 = i * 4 + j
                    tvec = jnp.broadcast_to(t2, (16,))
                    row = g * BBLK + t2
                    for h in range(2):
                        vec = (rows[row, pl.ds(h * LANES, LANES)] * SCALE
                               + poss[h])
                        plsc.store_scatter(
                            tile, [dgvec[h], gconst[g], drvec[h], tvec], vec)
                return 0

            lax.fori_loop(0, BBLK // 4, t_body, 0)

    for b in range(NBUF - 1):
        start_idx(b, b)
    for b in range(3):
        wait_idx(b, b)
        remap_idx(b)
        start_gather(b)

    def iter_body(it, _):
        for b in range(NBUF):
            u = it * NBUF + b
            wait_gather(b)
            compute(u, b)
            start_out(u, b)

            nb = (b + NBUF - 1) % NBUF

            @pl.when(u + NBUF - 1 < units_per_w)
            def _():
                @pl.when(u >= 1)
                def _():
                    wait_out(u - 1, nb)
                start_idx(u + NBUF - 1, nb)

            @pl.when(u + 3 < units_per_w)
            def _():
                wait_idx(u + 3, (b + 3) % NBUF)
                remap_idx((b + 3) % NBUF)
                start_gather((b + 3) % NBUF)
        return 0

    lax.fori_loop(0, units_per_w // NBUF, iter_body, 0)
    for k in range(NBUF):
        u = units_per_w - NBUF + k
        wait_out(u, u % NBUF)


def kernel(inputs, token_table, pos_table):
    B, S = inputs.shape
    assert S == SEQ and token_table.shape == (VOCAB, DIM)
    nbb = B // BBLK
    units = S * (B // UNIT)
    units_per_w = units // NW
    assert units_per_w % NBUF == 0

    idxt = inputs.T.astype(jnp.int32)
    tok_lin = _convert_table(token_table)

    mesh = plsc.VectorSubcoreMesh(core_axis_name="c", subcore_axis_name="s")
    run = functools.partial(
        pl.kernel,
        out_type=jax.ShapeDtypeStruct((SEQ, DIM // 8, nbb, 8, BBLK),
                                      jnp.float32),
        mesh=mesh,
        compiler_params=pltpu.CompilerParams(use_tc_tiling_on_sc=False,
                                             needs_layout_passes=False),
        scratch_types=[
            pltpu.VMEM((SEQ, DIM), jnp.float32),
            [pltpu.VMEM((UNIT,), jnp.int32) for _ in range(NBUF)],
            [pltpu.VMEM((UNIT, DIM), jnp.float32) for _ in range(NBUF)],
            [pltpu.VMEM((DIM // 8, GRP, 8, TPAD), jnp.float32)
             for _ in range(NBUF)],
            [pltpu.SemaphoreType.DMA for _ in range(NBUF)],
            [pltpu.SemaphoreType.DMA for _ in range(NBUF)],
            [pltpu.SemaphoreType.DMA for _ in range(NBUF)],
        ],
    )(functools.partial(_sc_body, units_per_w=units_per_w))

    out5 = run(idxt, tok_lin, pos_table)
    return jnp.transpose(out5, (2, 4, 0, 1, 3)).reshape(B, S, DIM)


# speedup vs baseline: 1.6444x; 1.6444x over previous
import functools
import math

import jax
import jax.numpy as jnp
from jax import lax
from jax.experimental import pallas as pl
from jax.experimental.pallas import tpu as pltpu
from jax.experimental.pallas import tpu_sc as plsc

SEQ = 200
DIM = 32
VOCAB = 1000000
LANES = 16
NC, NS = 2, 16
NW = NC * NS
SCALE = float(math.sqrt(float(DIM)))

QUART = 256000
R0 = 2048
NQB = QUART // R0
VPAD = 4 * QUART
BBLK = 128
GRP = 2
UNIT = GRP * BBLK
TPAD = 133
NBUF = 5


def _tc_transpose(t0, t1, t2, t3, out_ref):
    out_ref[...] = jnp.concatenate(
        [t0[...].T, t1[...].T, t2[...].T, t3[...].T], axis=1)


def _convert_table(token_table):
    tabt = token_table.T
    conv = pl.pallas_call(
        _tc_transpose,
        grid=(NQB,),
        in_specs=[
            pl.BlockSpec((DIM, R0), functools.partial(
                lambda j, i: (0, jnp.minimum(j * NQB + i, VOCAB // R0)), j))
            for j in range(4)
        ],
        out_specs=pl.BlockSpec((R0, 128), lambda i: (i, 0)),
        out_shape=jax.ShapeDtypeStruct((QUART, 128), jnp.float32),
    )(tabt, tabt, tabt, tabt)
    return conv.reshape(VPAD, DIM)


def _sc_body(idx_hbm, tok_hbm, pos_hbm, out_hbm,
             pos_v, idxbufs, rowbufs, tilebufs, isems, gsems, osems,
             *, units_per_w):
    wid = lax.axis_index("s") * NC + lax.axis_index("c")
    ubase = wid * units_per_w
    pltpu.sync_copy(pos_hbm, pos_v)
    iota16 = lax.iota(jnp.int32, 16)

    nq = 4096 // UNIT

    def unit_sb(u):
        g = ubase + u
        return g // nq, g % nq

    def start_idx(u, b):
        s, q = unit_sb(u)
        pltpu.async_copy(idx_hbm.at[s, pl.ds(q * UNIT, UNIT)], idxbufs[b],
                         isems[b])

    def wait_idx(u, b):
        s, q = unit_sb(u)
        pltpu.make_async_copy(idx_hbm.at[s, pl.ds(q * UNIT, UNIT)],
                              idxbufs[b], isems[b]).wait()

    def remap_idx(b):
        buf = idxbufs[b]
        for k in range(UNIT // LANES):
            v = buf[pl.ds(k * LANES, LANES)]
            vsub = ((v >= QUART).astype(jnp.int32)
                    + (v >= 2 * QUART).astype(jnp.int32)
                    + (v >= 3 * QUART).astype(jnp.int32))
            buf[pl.ds(k * LANES, LANES)] = (v - vsub * QUART) * 4 + vsub

    def start_gather(b):
        pltpu.async_copy(tok_hbm.at[idxbufs[b]], rowbufs[b], gsems[b])

    def wait_gather(b):
        pltpu.make_async_copy(tok_hbm.at[idxbufs[b]], rowbufs[b],
                              gsems[b]).wait()

    def start_out(u, b):
        s, q = unit_sb(u)
        pltpu.async_copy(tilebufs[b].at[:, :, :, pl.ds(0, BBLK)],
                         out_hbm.at[s, :, pl.ds(q * GRP, GRP)], osems[b])

    def wait_out(u, b):
        s, q = unit_sb(u)
        pltpu.make_async_copy(tilebufs[b].at[:, :, :, pl.ds(0, BBLK)],
                              out_hbm.at[s, :, pl.ds(q * GRP, GRP)],
                              osems[b]).wait()

    dvec = [iota16 + (h * LANES) for h in range(2)]
    dgvec = [d // 8 for d in dvec]
    drvec = [d % 8 for d in dvec]
    gconst = [jnp.broadcast_to(jnp.int32(g), (16,)) for g in range(GRP)]

    def compute(u, b):
        s, _ = unit_sb(u)
        rows = rowbufs[b]
        tile = tilebufs[b]
        poss = [pos_v[s, pl.ds(h * LANES, LANES)] for h in range(2)]

        for g in range(GRP):
            @plsc.parallel_loop(0, BBLK, step=1, unroll=4)
            def _(t2, g=g):
                tvec = jnp.broadcast_to(t2, (16,))
                row = g * BBLK + t2
                for h in range(2):
                    vec = (rows[row, pl.ds(h * LANES, LANES)] * SCALE
                           + poss[h])
                    plsc.store_scatter(
                        tile, [dgvec[h], gconst[g], drvec[h], tvec], vec)

    for b in range(NBUF - 1):
        start_idx(b, b)
    for b in range(3):
        wait_idx(b, b)
        remap_idx(b)
        start_gather(b)

    def iter_body(it, _):
        for b in range(NBUF):
            u = it * NBUF + b
            wait_gather(b)
            compute(u, b)
            start_out(u, b)

            nb = (b + NBUF - 1) % NBUF

            @pl.when(u + NBUF - 1 < units_per_w)
            def _():
                @pl.when(u >= 1)
                def _():
                    wait_out(u - 1, nb)
                start_idx(u + NBUF - 1, nb)

            @pl.when(u + 3 < units_per_w)
            def _():
                wait_idx(u + 3, (b + 3) % NBUF)
                remap_idx((b + 3) % NBUF)
                start_gather((b + 3) % NBUF)
        return 0

    lax.fori_loop(0, units_per_w // NBUF, iter_body, 0)
    for k in range(NBUF):
        u = units_per_w - NBUF + k
        wait_out(u, u % NBUF)


def kernel(inputs, token_table, pos_table):
    B, S = inputs.shape
    assert S == SEQ and token_table.shape == (VOCAB, DIM)
    nbb = B // BBLK
    units = S * (B // UNIT)
    units_per_w = units // NW
    assert units_per_w % NBUF == 0

    idxt = inputs.T.astype(jnp.int32)
    tok_lin = _convert_table(token_table)

    mesh = plsc.VectorSubcoreMesh(core_axis_name="c", subcore_axis_name="s")
    run = functools.partial(
        pl.kernel,
        out_type=jax.ShapeDtypeStruct((SEQ, DIM // 8, nbb, 8, BBLK),
                                      jnp.float32),
        mesh=mesh,
        compiler_params=pltpu.CompilerParams(use_tc_tiling_on_sc=False,
                                             needs_layout_passes=False),
        scratch_types=[
            pltpu.VMEM((SEQ, DIM), jnp.float32),
            [pltpu.VMEM((UNIT,), jnp.int32) for _ in range(NBUF)],
            [pltpu.VMEM((UNIT, DIM), jnp.float32) for _ in range(NBUF)],
            [pltpu.VMEM((DIM // 8, GRP, 8, TPAD), jnp.float32)
             for _ in range(NBUF)],
            [pltpu.SemaphoreType.DMA for _ in range(NBUF)],
            [pltpu.SemaphoreType.DMA for _ in range(NBUF)],
            [pltpu.SemaphoreType.DMA for _ in range(NBUF)],
        ],
    )(functools.partial(_sc_body, units_per_w=units_per_w))

    out5 = run(idxt, tok_lin, pos_table)
    return jnp.transpose(out5, (2, 4, 0, 1, 3)).reshape(B, S, DIM)


# speedup vs baseline: 1.7001x; 1.0339x over previous
import functools
import math

import jax
import jax.numpy as jnp
from jax import lax
from jax.experimental import pallas as pl
from jax.experimental.pallas import tpu as pltpu
from jax.experimental.pallas import tpu_sc as plsc

SEQ = 200
DIM = 32
VOCAB = 1000000
LANES = 16
NC, NS = 2, 16
NW = NC * NS
SCALE = float(math.sqrt(float(DIM)))

QUART = 253952
R0 = 4096
NQB = QUART // R0
VPAD = 4 * QUART
BBLK = 128
GRP = 2
UNIT = GRP * BBLK
TPAD = 133
NBUF = 5


def _tc_transpose(t0, t1, t2, t3, out_ref):
    out_ref[...] = jnp.concatenate(
        [t0[...].T, t1[...].T, t2[...].T, t3[...].T], axis=1)


def _convert_table(token_table):
    tabt = token_table.T
    conv = pl.pallas_call(
        _tc_transpose,
        grid=(NQB,),
        in_specs=[
            pl.BlockSpec((DIM, R0), functools.partial(
                lambda j, i: (0, jnp.minimum(j * NQB + i, VOCAB // R0)), j))
            for j in range(4)
        ],
        out_specs=pl.BlockSpec((R0, 128), lambda i: (i, 0)),
        out_shape=jax.ShapeDtypeStruct((QUART, 128), jnp.float32),
    )(tabt, tabt, tabt, tabt)
    return conv.reshape(VPAD, DIM)


def _sc_body(idx_hbm, tok_hbm, pos_hbm, out_hbm,
             pos_v, idxbufs, rowbufs, tilebufs, isems, gsems, osems,
             *, units_per_w):
    wid = lax.axis_index("s") * NC + lax.axis_index("c")
    ubase = wid * units_per_w
    pltpu.sync_copy(pos_hbm, pos_v)
    iota16 = lax.iota(jnp.int32, 16)

    nq = 4096 // UNIT

    def unit_sb(u):
        g = ubase + u
        return g // nq, g % nq

    def start_idx(u, b):
        s, q = unit_sb(u)
        pltpu.async_copy(idx_hbm.at[s, pl.ds(q * UNIT, UNIT)], idxbufs[b],
                         isems[b])

    def wait_idx(u, b):
        s, q = unit_sb(u)
        pltpu.make_async_copy(idx_hbm.at[s, pl.ds(q * UNIT, UNIT)],
                              idxbufs[b], isems[b]).wait()

    def remap_idx(b):
        buf = idxbufs[b]
        for k in range(UNIT // LANES):
            v = buf[pl.ds(k * LANES, LANES)]
            vsub = ((v >= QUART).astype(jnp.int32)
                    + (v >= 2 * QUART).astype(jnp.int32)
                    + (v >= 3 * QUART).astype(jnp.int32))
            buf[pl.ds(k * LANES, LANES)] = (v - vsub * QUART) * 4 + vsub

    def start_gather(b):
        pltpu.async_copy(tok_hbm.at[idxbufs[b]], rowbufs[b], gsems[b])

    def wait_gather(b):
        pltpu.make_async_copy(tok_hbm.at[idxbufs[b]], rowbufs[b],
                              gsems[b]).wait()

    def start_out(u, b):
        s, q = unit_sb(u)
        pltpu.async_copy(tilebufs[b].at[:, :, :, pl.ds(0, BBLK)],
                         out_hbm.at[s, :, pl.ds(q * GRP, GRP)], osems[b])

    def wait_out(u, b):
        s, q = unit_sb(u)
        pltpu.make_async_copy(tilebufs[b].at[:, :, :, pl.ds(0, BBLK)],
                              out_hbm.at[s, :, pl.ds(q * GRP, GRP)],
                              osems[b]).wait()

    dvec = [iota16 + (h * LANES) for h in range(2)]
    dgvec = [d // 8 for d in dvec]
    drvec = [d % 8 for d in dvec]
    gconst = [jnp.broadcast_to(jnp.int32(g), (16,)) for g in range(GRP)]

    def compute(u, b):
        s, _ = unit_sb(u)
        rows = rowbufs[b]
        tile = tilebufs[b]
        poss = [pos_v[s, pl.ds(h * LANES, LANES)] for h in range(2)]

        for g in range(GRP):
            @plsc.parallel_loop(0, BBLK, step=1, unroll=4)
            def _(t2, g=g):
                tvec = jnp.broadcast_to(t2, (16,))
                row = g * BBLK + t2
                for h in range(2):
                    vec = (rows[row, pl.ds(h * LANES, LANES)] * SCALE
                           + poss[h])
                    plsc.store_scatter(
                        tile, [dgvec[h], gconst[g], drvec[h], tvec], vec)

    for b in range(NBUF - 1):
        start_idx(b, b)
    for b in range(3):
        wait_idx(b, b)
        remap_idx(b)
        start_gather(b)

    def iter_body(it, _):
        for b in range(NBUF):
            u = it * NBUF + b
            wait_gather(b)
            compute(u, b)
            start_out(u, b)

            nb = (b + NBUF - 1) % NBUF

            @pl.when(u + NBUF - 1 < units_per_w)
            def _():
                @pl.when(u >= 1)
                def _():
                    wait_out(u - 1, nb)
                start_idx(u + NBUF - 1, nb)

            @pl.when(u + 3 < units_per_w)
            def _():
                wait_idx(u + 3, (b + 3) % NBUF)
                remap_idx((b + 3) % NBUF)
                start_gather((b + 3) % NBUF)
        return 0

    lax.fori_loop(0, units_per_w // NBUF, iter_body, 0)
    for k in range(NBUF):
        u = units_per_w - NBUF + k
        wait_out(u, u % NBUF)


def kernel(inputs, token_table, pos_table):
    B, S = inputs.shape
    assert S == SEQ and token_table.shape == (VOCAB, DIM)
    nbb = B // BBLK
    units = S * (B // UNIT)
    units_per_w = units // NW
    assert units_per_w % NBUF == 0

    idxt = inputs.T.astype(jnp.int32)
    tok_lin = _convert_table(token_table)

    mesh = plsc.VectorSubcoreMesh(core_axis_name="c", subcore_axis_name="s")
    run = functools.partial(
        pl.kernel,
        out_type=jax.ShapeDtypeStruct((SEQ, DIM // 8, nbb, 8, BBLK),
                                      jnp.float32),
        mesh=mesh,
        compiler_params=pltpu.CompilerParams(use_tc_tiling_on_sc=False,
                                             needs_layout_passes=False),
        scratch_types=[
            pltpu.VMEM((SEQ, DIM), jnp.float32),
            [pltpu.VMEM((UNIT,), jnp.int32) for _ in range(NBUF)],
            [pltpu.VMEM((UNIT, DIM), jnp.float32) for _ in range(NBUF)],
            [pltpu.VMEM((DIM // 8, GRP, 8, TPAD), jnp.float32)
             for _ in range(NBUF)],
            [pltpu.SemaphoreType.DMA for _ in range(NBUF)],
            [pltpu.SemaphoreType.DMA for _ in range(NBUF)],
            [pltpu.SemaphoreType.DMA for _ in range(NBUF)],
        ],
    )(functools.partial(_sc_body, units_per_w=units_per_w))

    out5 = run(idxt, tok_lin, pos_table)
    return jnp.transpose(out5, (2, 4, 0, 1, 3)).reshape(B, S, DIM)
